# baseline (speedup 1.0000x reference)
import functools

import jax
import jax.numpy as jnp
from jax.experimental import pallas as pl
from jax.experimental.pallas import tpu as pltpu

H = 128
L = 50


def _gnn_block(num_ref, node_ref, glob_ref, un_ref,
               w2t_ref, b2_ref, w1_ref, b1_ref, w5t_ref, b5_ref,
               out_ref, *, sb):
    nb = sb * L
    node = node_ref[:]

    w2t = w2t_ref[:]
    f32 = jnp.float32
    bf16 = jnp.bfloat16
    w2t_b = w2t.astype(bf16)
    node_b = node.astype(bf16)

    pre = jnp.dot(node_b, w2t_b[H:2 * H], preferred_element_type=f32)
    pre += jnp.dot(glob_ref[:].astype(bf16), w2t_b[2 * H:3 * H],
                   preferred_element_type=f32)
    pre += jnp.dot(un_ref[:].astype(bf16), w2t_b[3 * H:4 * H],
                   preferred_element_type=f32)
    pre += b2_ref[:]

    iota_s = jax.lax.broadcasted_iota(jnp.int32, (sb, nb), 0)
    iota_n = jax.lax.broadcasted_iota(jnp.int32, (sb, nb), 1)
    sel = (iota_n == iota_s * L + (L - 1)).astype(f32)
    seg = (iota_n // L == iota_s).astype(bf16)

    v_last = jnp.dot(sel, node, preferred_element_type=f32)
    vl_proj = jnp.dot(v_last, w2t[0:H], preferred_element_type=f32)

    pre += jnp.dot(seg.T, vl_proj.astype(bf16), preferred_element_type=f32)

    s = jnp.tanh(pre)
    alpha = jnp.sum(s * w1_ref[:], axis=1, keepdims=True) + b1_ref[0, 0]
    weighted = (num_ref[:] * alpha) * node

    s_g = jnp.dot(seg, weighted.astype(bf16), preferred_element_type=f32)

    w5t = w5t_ref[:]
    out = jnp.dot(v_last, w5t[0:H], preferred_element_type=f32)
    out += jnp.dot(s_g, w5t[H:2 * H], preferred_element_type=f32)
    out += b5_ref[:]
    out_ref[:] = out


@functools.partial(jax.jit, static_argnames=("sb",))
def _run(node_embedding, global_node_embedding, num_count, u_n_repeat,
         W1_w, W1_b, W2_w, W2_b, W5_w, W5_b, sb=64):
    n, h = node_embedding.shape
    b = n // L
    nb = sb * L
    grid = (b // sb,)

    num2 = num_count.reshape(n, 1)
    w2t = W2_w.T * 0.5
    w5t = W5_w.T
    b2 = W2_b.reshape(1, H) * 0.5
    b5 = W5_b.reshape(1, H)
    w1 = W1_w * 0.5
    b1 = (0.5 * jnp.sum(W1_w) + W1_b).reshape(1, 1)

    row_spec = pl.BlockSpec((nb, h), lambda i: (i, 0))
    full = lambda a: pl.BlockSpec(a.shape, lambda i: (0,) * a.ndim)

    return pl.pallas_call(
        functools.partial(_gnn_block, sb=sb),
        grid=grid,
        in_specs=[
            pl.BlockSpec((nb, 1), lambda i: (i, 0)),
            row_spec, row_spec, row_spec,
            full(w2t), full(b2), full(w1), full(b1), full(w5t), full(b5),
        ],
        out_specs=pl.BlockSpec((sb, h), lambda i: (i, 0)),
        out_shape=jax.ShapeDtypeStruct((b, h), jnp.float32),
        compiler_params=pltpu.CompilerParams(
            dimension_semantics=("parallel",)),
    )(num2, node_embedding, global_node_embedding, u_n_repeat,
      w2t, b2, w1, b1, w5t, b5)


def kernel(node_embedding, global_node_embedding, item_embedding_table,
           sections, num_count, user_embedding, max_item_id, u_n_repeat,
           W1_w, W1_b, W2_w, W2_b, W5_w, W5_b):
    return _run(node_embedding, global_node_embedding, num_count, u_n_repeat,
                W1_w, W1_b, W2_w, W2_b, W5_w, W5_b)


# speedup vs baseline: 1.6133x; 1.6133x over previous
import functools

import jax
import jax.numpy as jnp
from jax.experimental import pallas as pl
from jax.experimental.pallas import tpu as pltpu

H = 128
L = 50


def _gnn_block(num_ref, node_ref, glob_ref, un_ref,
               w2t_ref, b2_ref, w1rep_ref, b1_ref, w5t_ref, b5_ref,
               out_ref, *, sb):
    nb = sb * L
    node = node_ref[:]

    w2t = w2t_ref[:]
    f32 = jnp.float32
    bf16 = jnp.bfloat16
    w2t_b = w2t.astype(bf16)
    node_b = node.astype(bf16)

    pre = jnp.dot(node_b, w2t_b[H:2 * H], preferred_element_type=f32)
    pre += jnp.dot(glob_ref[:].astype(bf16), w2t_b[2 * H:3 * H],
                   preferred_element_type=f32)
    pre += jnp.dot(un_ref[:].astype(bf16), w2t_b[3 * H:4 * H],
                   preferred_element_type=f32)
    pre += b2_ref[:]

    iota_s = jax.lax.broadcasted_iota(jnp.int32, (sb, nb), 0)
    iota_n = jax.lax.broadcasted_iota(jnp.int32, (sb, nb), 1)
    sel = (iota_n == iota_s * L + (L - 1)).astype(bf16)
    seg = (iota_n // L == iota_s)
    segw = jnp.where(seg, num_ref[:], 0.0).astype(bf16)

    v_last = jnp.dot(sel, node_b, preferred_element_type=f32)
    vl_proj = jnp.dot(v_last, w2t[0:H], preferred_element_type=f32)

    pre += jnp.dot(seg.astype(bf16).T, vl_proj.astype(bf16),
                   preferred_element_type=f32)

    s = jnp.tanh(pre)
    alpha = jnp.dot(s.astype(bf16), w1rep_ref[:],
                    preferred_element_type=f32) + b1_ref[0, 0]
    weighted = alpha * node

    s_g = jnp.dot(segw, weighted.astype(bf16), preferred_element_type=f32)

    w5t = w5t_ref[:]
    out = jnp.dot(v_last, w5t[0:H], preferred_element_type=f32)
    out += jnp.dot(s_g, w5t[H:2 * H], preferred_element_type=f32)
    out += b5_ref[:]
    out_ref[:] = out


@functools.partial(jax.jit, static_argnames=("sb",))
def _run(node_embedding, global_node_embedding, num_count, u_n_repeat,
         W1_w, W1_b, W2_w, W2_b, W5_w, W5_b, sb=128):
    n, h = node_embedding.shape
    b = n // L
    nb = sb * L
    grid = (b // sb,)

    num2 = num_count.reshape(1, n)
    w2t = W2_w.T * 0.5
    w5t = W5_w.T
    b2 = W2_b.reshape(1, H) * 0.5
    b5 = W5_b.reshape(1, H)
    w1rep = jnp.tile((W1_w * 0.5).reshape(H, 1), (1, H)).astype(jnp.bfloat16)
    b1 = (0.5 * jnp.sum(W1_w) + W1_b).reshape(1, 1)

    row_spec = pl.BlockSpec((nb, h), lambda i: (i, 0))
    full = lambda a: pl.BlockSpec(a.shape, lambda i: (0,) * a.ndim)

    return pl.pallas_call(
        functools.partial(_gnn_block, sb=sb),
        grid=grid,
        in_specs=[
            pl.BlockSpec((1, nb), lambda i: (0, i)),
            row_spec, row_spec, row_spec,
            full(w2t), full(b2), full(w1rep), full(b1), full(w5t), full(b5),
        ],
        out_specs=pl.BlockSpec((sb, h), lambda i: (i, 0)),
        out_shape=jax.ShapeDtypeStruct((b, h), jnp.float32),
        compiler_params=pltpu.CompilerParams(
            dimension_semantics=("parallel",)),
    )(num2, node_embedding, global_node_embedding, u_n_repeat,
      w2t, b2, w1rep, b1, w5t, b5)


def kernel(node_embedding, global_node_embedding, item_embedding_table,
           sections, num_count, user_embedding, max_item_id, u_n_repeat,
           W1_w, W1_b, W2_w, W2_b, W5_w, W5_b):
    return _run(node_embedding, global_node_embedding, num_count, u_n_repeat,
                W1_w, W1_b, W2_w, W2_b, W5_w, W5_b)
